# Initial kernel scaffold; baseline (speedup 1.0000x reference)
import jax
import jax.numpy as jnp
from jax.experimental import pallas as pl


def kernel(x_m, xm_cmp, q_w, km_cmp):
    raise NotImplementedError("write your pallas kernel here")


# speedup vs baseline: 2.9711x; 2.9711x over previous
import functools

import jax
import jax.numpy as jnp
from jax import lax
from jax.experimental import pallas as pl
from jax.experimental.pallas import tpu as pltpu
from jax.experimental.pallas import tpu_sc as plsc

HEAD_DIM = 128
R_SEL = 0.25
CHUNK = 256
SC_CORES = 2
SC_SUBCORES = 16
SC_WORKERS = SC_CORES * SC_SUBCORES
ROWS_PER_DMA = 16


def _imp_body(q_ref, k_ref, out_ref):
    h = pl.program_id(1)
    q = q_ref[0, 0]
    k = k_ref[0, 0]
    s = lax.dot_general(
        q, k, (((1,), (1,)), ((), ())),
        preferred_element_type=jnp.float32,
    ) * (HEAD_DIM ** -0.5)
    m = jnp.max(s, axis=1, keepdims=True)
    e = jnp.exp(s - m)
    d = jnp.sum(e, axis=1, keepdims=True)
    contrib = jnp.sum(e / d, axis=0)[None, None, :]

    @pl.when(h == 0)
    def _init():
        out_ref[...] = contrib

    @pl.when(h != 0)
    def _acc():
        out_ref[...] = out_ref[...] + contrib


def _sel_body(num_sel, out_len, t2, imp_ref, impT_ref, dsta_ref, srcb_ref,
              dstb_ref):
    b = pl.program_id(0)
    T = imp_ref.shape[2]
    nch = T // CHUNK
    v_row = imp_ref[0]
    t_row = lax.broadcasted_iota(jnp.int32, (1, T), 1).astype(jnp.float32)

    rank_row = jnp.zeros((1, T), jnp.float32)
    rank_cols = []
    for ci in range(nch):
        vu = impT_ref[0, ci * CHUNK:(ci + 1) * CHUNK, :]
        u_col = lax.broadcasted_iota(jnp.int32, (CHUNK, 1), 0).astype(jnp.float32) + ci * CHUNK
        beats = (vu > v_row) | ((vu == v_row) & (u_col < t_row))
        bf = beats.astype(jnp.float32)
        rank_row = rank_row + jnp.sum(bf, axis=0, keepdims=True)
        rank_cols.append((T - 1.0) - jnp.sum(bf, axis=1, keepdims=True))
    mask_row = rank_row < num_sel
    maskf_row = mask_row.astype(jnp.float32)

    j_row = lax.broadcasted_iota(jnp.int32, (1, num_sel), 1).astype(jnp.float32)
    src_acc = jnp.zeros((1, num_sel), jnp.float32)
    dst_acc = jnp.zeros((1, num_sel), jnp.float32)
    nsel_row = jnp.zeros((1, T), jnp.float32)
    for ci in range(nch):
        u_col = lax.broadcasted_iota(jnp.int32, (CHUNK, 1), 0).astype(jnp.float32) + ci * CHUNK
        maskf_col = (rank_cols[ci] < num_sel).astype(jnp.float32)
        nsel_col = jnp.sum(maskf_row * (t_row < u_col).astype(jnp.float32),
                           axis=1, keepdims=True)
        nsel_row = nsel_row + jnp.sum(
            maskf_col * (u_col < t_row).astype(jnp.float32),
            axis=0, keepdims=True)
        pos_col = u_col + nsel_col
        oh = maskf_col * (nsel_col == j_row).astype(jnp.float32)
        src_acc = src_acc + jnp.sum(oh * u_col, axis=0, keepdims=True)
        dst_acc = dst_acc + jnp.sum(oh * pos_col, axis=0, keepdims=True)

    ybase = b * (out_len + 1)
    pos_row = (t_row + nsel_row).astype(jnp.int32) + ybase
    junk = ybase + out_len
    dsta_ref[0] = jnp.where(mask_row, junk, pos_row)
    srcb_ref[0] = 2 * src_acc.astype(jnp.int32) + b * t2
    dstb_ref[0] = dst_acc.astype(jnp.int32) + ybase


def _make_sc_interleave(B, T, C, num_sel, out_len):
    a_per_w = B * T // SC_WORKERS
    b_per_w = B * num_sel // SC_WORKERS
    nca = a_per_w // ROWS_PER_DMA
    ncb = b_per_w // ROWS_PER_DMA
    mesh = plsc.VectorSubcoreMesh(core_axis_name="c", subcore_axis_name="s")

    @functools.partial(
        pl.kernel,
        mesh=mesh,
        out_type=jax.ShapeDtypeStruct((B * (out_len + 1), C), jnp.float32),
        scratch_types=[
            pltpu.VMEM((ROWS_PER_DMA,), jnp.int32),
            pltpu.VMEM((ROWS_PER_DMA,), jnp.int32),
            pltpu.VMEM((ROWS_PER_DMA, C), jnp.float32),
            pltpu.SemaphoreType.DMA,
        ],
    )
    def sc_fn(xmc, xm, dsta, srcb, dstb, y, idx1, idx2, rows, sem):
        wid = lax.axis_index("s") * SC_CORES + lax.axis_index("c")
        for j in range(nca):
            base = wid * a_per_w + j * ROWS_PER_DMA
            pltpu.sync_copy(dsta.at[pl.ds(base, ROWS_PER_DMA)], idx1)
            pltpu.sync_copy(xmc.at[pl.ds(base, ROWS_PER_DMA)], rows)
            pltpu.async_copy(rows, y.at[idx1], sem).wait()
        for j in range(ncb):
            base = wid * b_per_w + j * ROWS_PER_DMA
            pltpu.sync_copy(srcb.at[pl.ds(base, ROWS_PER_DMA)], idx1)
            pltpu.sync_copy(dstb.at[pl.ds(base, ROWS_PER_DMA)], idx2)
            pltpu.async_copy(xm.at[idx1], rows, sem).wait()
            pltpu.async_copy(rows, y.at[idx2], sem).wait()
            idx1[...] = idx1[...] + 1
            idx2[...] = idx2[...] + 1
            pltpu.async_copy(xm.at[idx1], rows, sem).wait()
            pltpu.async_copy(rows, y.at[idx2], sem).wait()

    return sc_fn


def kernel(x_m, xm_cmp, q_w, km_cmp):
    B, T, C = xm_cmp.shape
    H = q_w.shape[1]
    KV = km_cmp.shape[1]
    groups = H // KV
    Tq = q_w.shape[2]
    D = q_w.shape[3]
    num_sel = int(R_SEL * T)
    out_len = T + num_sel

    imp = pl.pallas_call(
        _imp_body,
        grid=(B, H),
        in_specs=[
            pl.BlockSpec((1, 1, Tq, D), lambda b, h: (b, h, 0, 0)),
            pl.BlockSpec((1, 1, T, D), lambda b, h: (b, h // groups, 0, 0)),
        ],
        out_specs=pl.BlockSpec((1, 1, T), lambda b, h: (b, 0, 0)),
        out_shape=jax.ShapeDtypeStruct((B, 1, T), jnp.float32),
        compiler_params=pltpu.CompilerParams(
            dimension_semantics=("parallel", "arbitrary")),
    )(q_w, km_cmp)

    impT = imp.reshape(B, T, 1)
    dsta, srcb, dstb = pl.pallas_call(
        functools.partial(_sel_body, num_sel, out_len, 2 * T),
        grid=(B,),
        in_specs=[
            pl.BlockSpec((1, 1, T), lambda b: (b, 0, 0)),
            pl.BlockSpec((1, T, 1), lambda b: (b, 0, 0)),
        ],
        out_specs=[
            pl.BlockSpec((1, 1, T), lambda b: (b, 0, 0)),
            pl.BlockSpec((1, 1, num_sel), lambda b: (b, 0, 0)),
            pl.BlockSpec((1, 1, num_sel), lambda b: (b, 0, 0)),
        ],
        out_shape=[
            jax.ShapeDtypeStruct((B, 1, T), jnp.int32),
            jax.ShapeDtypeStruct((B, 1, num_sel), jnp.int32),
            jax.ShapeDtypeStruct((B, 1, num_sel), jnp.int32),
        ],
    )(imp, impT)

    sc_fn = _make_sc_interleave(B, T, C, num_sel, out_len)
    y = sc_fn(
        xm_cmp.reshape(B * T, C),
        x_m.reshape(B * 2 * T, C),
        dsta.reshape(B * T),
        srcb.reshape(B * num_sel),
        dstb.reshape(B * num_sel),
    )
    return y.reshape(B, out_len + 1, C)[:, :out_len]


# speedup vs baseline: 4.9550x; 1.6678x over previous
import functools

import jax
import jax.numpy as jnp
from jax import lax
from jax.experimental import pallas as pl
from jax.experimental.pallas import tpu as pltpu
from jax.experimental.pallas import tpu_sc as plsc

HEAD_DIM = 128
R_SEL = 0.25
CHUNK = 256
SC_CORES = 2
SC_SUBCORES = 16
SC_WORKERS = SC_CORES * SC_SUBCORES
ROWS_PER_DMA = 16


def _imp_body(q_ref, k_ref, out_ref):
    h = pl.program_id(1)
    q = q_ref[0, 0]
    k = k_ref[0, 0]
    s = lax.dot_general(
        q, k, (((1,), (1,)), ((), ())),
        preferred_element_type=jnp.float32,
    ) * (HEAD_DIM ** -0.5)
    m = jnp.max(s, axis=1, keepdims=True)
    e = jnp.exp(s - m)
    d = jnp.sum(e, axis=1, keepdims=True)
    contrib = jnp.sum(e / d, axis=0)[None, None, :]

    @pl.when(h == 0)
    def _init():
        out_ref[...] = contrib

    @pl.when(h != 0)
    def _acc():
        out_ref[...] = out_ref[...] + contrib


def _sel_body(num_sel, out_len, t2, imp_ref, impT_ref, srca_ref, dsta_ref,
              srcb_ref, dstb_ref):
    b = pl.program_id(0)
    T = imp_ref.shape[2]
    num_unsel = T - num_sel
    nch = T // CHUNK
    v_row = imp_ref[0]
    t_row = lax.broadcasted_iota(jnp.int32, (1, T), 1).astype(jnp.float32)

    rank_row = jnp.zeros((1, T), jnp.float32)
    rank_cols = []
    for ci in range(nch):
        vu = impT_ref[0, ci * CHUNK:(ci + 1) * CHUNK, :]
        u_col = lax.broadcasted_iota(
            jnp.int32, (CHUNK, 1), 0).astype(jnp.float32) + ci * CHUNK
        beats = (vu > v_row) | ((vu == v_row) & (u_col < t_row))
        bf = beats.astype(jnp.float32)
        rank_row = rank_row + jnp.sum(bf, axis=0, keepdims=True)
        rank_cols.append((T - 1.0) - jnp.sum(bf, axis=1, keepdims=True))
    mask_row = rank_row < num_sel
    maskf_row = mask_row.astype(jnp.float32)

    js_row = lax.broadcasted_iota(
        jnp.int32, (1, num_sel), 1).astype(jnp.float32)
    ju_row = lax.broadcasted_iota(
        jnp.int32, (1, num_unsel), 1).astype(jnp.float32)
    sel_src = jnp.zeros((1, num_sel), jnp.float32)
    sel_dst = jnp.zeros((1, num_sel), jnp.float32)
    uns_src = jnp.zeros((1, num_unsel), jnp.float32)
    uns_dst = jnp.zeros((1, num_unsel), jnp.float32)
    for ci in range(nch):
        u_col = lax.broadcasted_iota(
            jnp.int32, (CHUNK, 1), 0).astype(jnp.float32) + ci * CHUNK
        maskf_col = (rank_cols[ci] < num_sel).astype(jnp.float32)
        nsel_col = jnp.sum(maskf_row * (t_row < u_col).astype(jnp.float32),
                           axis=1, keepdims=True)
        pos_col = u_col + nsel_col
        oh_s = maskf_col * (nsel_col == js_row).astype(jnp.float32)
        sel_src = sel_src + jnp.sum(oh_s * u_col, axis=0, keepdims=True)
        sel_dst = sel_dst + jnp.sum(oh_s * pos_col, axis=0, keepdims=True)
        nuns_col = u_col - nsel_col
        oh_u = (1.0 - maskf_col) * (nuns_col == ju_row).astype(jnp.float32)
        uns_src = uns_src + jnp.sum(oh_u * u_col, axis=0, keepdims=True)
        uns_dst = uns_dst + jnp.sum(oh_u * pos_col, axis=0, keepdims=True)

    ybase = b * out_len
    srca_ref[0] = uns_src.astype(jnp.int32) + b * T
    dsta_ref[0] = uns_dst.astype(jnp.int32) + ybase
    sel_src_i = sel_src.astype(jnp.int32)
    sel_dst_i = sel_dst.astype(jnp.int32)
    srcb_ref[0, :, :num_sel] = 2 * sel_src_i + b * t2
    srcb_ref[0, :, num_sel:] = 2 * sel_src_i + 1 + b * t2
    dstb_ref[0, :, :num_sel] = sel_dst_i + ybase
    dstb_ref[0, :, num_sel:] = sel_dst_i + 1 + ybase


def _make_sc_interleave(B, T, C, num_sel, out_len):
    num_unsel = T - num_sel
    a_rows = B * num_unsel // SC_WORKERS
    b_rows = 2 * B * num_sel // SC_WORKERS
    nca = a_rows // ROWS_PER_DMA
    ncb = b_rows // ROWS_PER_DMA
    mesh = plsc.VectorSubcoreMesh(core_axis_name="c", subcore_axis_name="s")

    @functools.partial(
        pl.kernel,
        mesh=mesh,
        out_type=jax.ShapeDtypeStruct((B * out_len, C), jnp.float32),
        scratch_types=[
            pltpu.VMEM((nca, ROWS_PER_DMA), jnp.int32),
            pltpu.VMEM((nca, ROWS_PER_DMA), jnp.int32),
            pltpu.VMEM((ncb, ROWS_PER_DMA), jnp.int32),
            pltpu.VMEM((ncb, ROWS_PER_DMA), jnp.int32),
            pltpu.VMEM((ROWS_PER_DMA, C), jnp.float32),
            pltpu.VMEM((ROWS_PER_DMA, C), jnp.float32),
            pltpu.SemaphoreType.DMA,
            pltpu.SemaphoreType.DMA,
            pltpu.SemaphoreType.DMA,
            pltpu.SemaphoreType.DMA,
            pltpu.SemaphoreType.DMA,
        ],
    )
    def sc_fn(xmc, xm, srca, dsta, srcb, dstb, y,
              sia, dia, sib, dib, rows0, rows1,
              gsem0, gsem1, ssem0, ssem1, isem):
        wid = lax.axis_index("s") * SC_CORES + lax.axis_index("c")
        ph = [
            pltpu.async_copy(srca.at[wid], sia, isem),
            pltpu.async_copy(dsta.at[wid], dia, isem),
            pltpu.async_copy(srcb.at[wid], sib, isem),
            pltpu.async_copy(dstb.at[wid], dib, isem),
        ]
        for h in ph:
            h.wait()
        work = [(xmc, sia, dia, j) for j in range(nca)]
        work += [(xm, sib, dib, j) for j in range(ncb)]
        bufs = [(rows0, gsem0, ssem0), (rows1, gsem1, ssem1)]
        n = len(work)
        handles = [None] * n
        for i, (src, si, di, j) in enumerate(work):
            rows, gsem, ssem = bufs[i % 2]
            if i >= 2:
                handles[i - 2].wait()
            pltpu.async_copy(src.at[si.at[j]], rows, gsem).wait()
            handles[i] = pltpu.async_copy(rows, y.at[di.at[j]], ssem)
        handles[n - 2].wait()
        handles[n - 1].wait()

    return sc_fn


def kernel(x_m, xm_cmp, q_w, km_cmp):
    B, T, C = xm_cmp.shape
    H = q_w.shape[1]
    KV = km_cmp.shape[1]
    groups = H // KV
    Tq = q_w.shape[2]
    D = q_w.shape[3]
    num_sel = int(R_SEL * T)
    num_unsel = T - num_sel
    out_len = T + num_sel

    imp = pl.pallas_call(
        _imp_body,
        grid=(B, H),
        in_specs=[
            pl.BlockSpec((1, 1, Tq, D), lambda b, h: (b, h, 0, 0)),
            pl.BlockSpec((1, 1, T, D), lambda b, h: (b, h // groups, 0, 0)),
        ],
        out_specs=pl.BlockSpec((1, 1, T), lambda b, h: (b, 0, 0)),
        out_shape=jax.ShapeDtypeStruct((B, 1, T), jnp.float32),
        compiler_params=pltpu.CompilerParams(
            dimension_semantics=("parallel", "arbitrary")),
    )(q_w, km_cmp)

    impT = imp.reshape(B, T, 1)
    srca, dsta, srcb, dstb = pl.pallas_call(
        functools.partial(_sel_body, num_sel, out_len, 2 * T),
        grid=(B,),
        in_specs=[
            pl.BlockSpec((1, 1, T), lambda b: (b, 0, 0)),
            pl.BlockSpec((1, T, 1), lambda b: (b, 0, 0)),
        ],
        out_specs=[
            pl.BlockSpec((1, 1, num_unsel), lambda b: (b, 0, 0)),
            pl.BlockSpec((1, 1, num_unsel), lambda b: (b, 0, 0)),
            pl.BlockSpec((1, 1, 2 * num_sel), lambda b: (b, 0, 0)),
            pl.BlockSpec((1, 1, 2 * num_sel), lambda b: (b, 0, 0)),
        ],
        out_shape=[
            jax.ShapeDtypeStruct((B, 1, num_unsel), jnp.int32),
            jax.ShapeDtypeStruct((B, 1, num_unsel), jnp.int32),
            jax.ShapeDtypeStruct((B, 1, 2 * num_sel), jnp.int32),
            jax.ShapeDtypeStruct((B, 1, 2 * num_sel), jnp.int32),
        ],
    )(imp, impT)

    sc_fn = _make_sc_interleave(B, T, C, num_sel, out_len)
    y = sc_fn(
        xm_cmp.reshape(B * T, C),
        x_m.reshape(B * 2 * T, C),
        srca.reshape(SC_WORKERS, -1, ROWS_PER_DMA),
        dsta.reshape(SC_WORKERS, -1, ROWS_PER_DMA),
        srcb.reshape(SC_WORKERS, -1, ROWS_PER_DMA),
        dstb.reshape(SC_WORKERS, -1, ROWS_PER_DMA),
    )
    return y.reshape(B, out_len, C)
